# Initial kernel scaffold; baseline (speedup 1.0000x reference)
import jax
import jax.numpy as jnp
from jax.experimental import pallas as pl


def kernel(x, W_enc, b_enc, W_dec, b_dec):
    raise NotImplementedError("write your pallas kernel here")


# speedup vs baseline: 22.1074x; 22.1074x over previous
import jax
import jax.numpy as jnp
from jax import lax
from jax.experimental import pallas as pl
from jax.experimental.pallas import tpu as pltpu

_K_PER_TOKEN = 64
_NPROBE = 16
_NPASS = 8


def _encode_body(x_ref, w_ref, be_ref, bd_ref, post_ref, max_ref):
    xc = x_ref[...] - bd_ref[...]
    acc = lax.dot_general(xc, w_ref[...], (((1,), (1,)), ((), ())),
                          preferred_element_type=jnp.float32)
    post = jnp.maximum(acc + be_ref[...][0], 0.0)
    post_ref[...] = post
    max_ref[...] = jnp.full(max_ref.shape, jnp.max(post), dtype=jnp.float32)


def _select_body(k_total, n_chunks, maxb_ref, post_ref, theta_ref,
                 state_ref, counts_ref):
    p = pl.program_id(0)
    c = pl.program_id(1)

    @pl.when((p == 0) & (c == 0))
    def _init():
        state_ref[0] = -1
        state_ref[1] = maxb_ref[0, 0]

    @pl.when(c == 0)
    def _zero():
        for j in range(_NPROBE):
            counts_ref[j] = 0

    lo = state_ref[0]
    hi = state_ref[1]
    step = jnp.maximum((hi - lo + (_NPROBE - 1)) // _NPROBE, 1)
    bits = lax.bitcast_convert_type(post_ref[...], jnp.int32)
    for j in range(_NPROBE):
        t = jnp.minimum(lo + step * (j + 1), hi)
        counts_ref[j] += jnp.sum((bits > t).astype(jnp.int32))

    @pl.when(c == n_chunks - 1)
    def _update():
        jstar = 0
        for j in range(_NPROBE):
            jstar = jstar + (counts_ref[j] >= k_total).astype(jnp.int32)
        new_lo = jnp.minimum(lo + step * jstar, hi)
        new_hi = jnp.minimum(lo + step * (jstar + 1), hi)
        state_ref[0] = new_lo
        state_ref[1] = new_hi

    theta_ref[...] = jnp.full((1, 1), state_ref[1], dtype=jnp.int32)


def _decode_body(theta_ref, post_ref, w_ref, bd_ref, out_ref):
    th = lax.bitcast_convert_type(theta_ref[0, 0], jnp.float32)
    post = post_ref[...]
    enc = jnp.where(post >= th, post, 0.0)
    acc = lax.dot_general(enc, w_ref[...], (((1,), (1,)), ((), ())),
                          preferred_element_type=jnp.float32)
    out_ref[...] = acc + bd_ref[...]


def kernel(x, W_enc, b_enc, W_dec, b_dec):
    n_tokens, act_dim = x.shape
    dict_size = W_enc.shape[0]
    k_total = _K_PER_TOKEN * n_tokens

    bn = min(1024, dict_size)
    n_blocks = dict_size // bn
    ch = min(256, n_tokens)
    n_chunks = n_tokens // ch
    bm = min(256, n_tokens)
    m_blocks = n_tokens // bm

    bd2 = b_dec.reshape(1, act_dim)
    be3 = b_enc.reshape(n_blocks, 1, bn)

    post, maxes = pl.pallas_call(
        _encode_body,
        grid=(n_blocks,),
        in_specs=[
            pl.BlockSpec((n_tokens, act_dim), lambda n: (0, 0)),
            pl.BlockSpec((bn, act_dim), lambda n: (n, 0)),
            pl.BlockSpec((1, 1, bn), lambda n: (n, 0, 0)),
            pl.BlockSpec((1, act_dim), lambda n: (0, 0)),
        ],
        out_specs=[
            pl.BlockSpec((n_tokens, bn), lambda n: (0, n)),
            pl.BlockSpec((1, 1, 128), lambda n: (n, 0, 0)),
        ],
        out_shape=[
            jax.ShapeDtypeStruct((n_tokens, dict_size), jnp.float32),
            jax.ShapeDtypeStruct((n_blocks, 1, 128), jnp.float32),
        ],
    )(x, W_enc, be3, bd2)

    maxbits = lax.bitcast_convert_type(jnp.max(maxes), jnp.int32)
    maxbits = maxbits.reshape(1, 1)

    theta_bits = pl.pallas_call(
        lambda *refs: _select_body(k_total, n_chunks, *refs),
        grid=(_NPASS, n_chunks),
        in_specs=[
            pl.BlockSpec(memory_space=pltpu.SMEM),
            pl.BlockSpec((ch, dict_size), lambda p, c: (c, 0)),
        ],
        out_specs=pl.BlockSpec((1, 1), lambda p, c: (0, 0)),
        out_shape=jax.ShapeDtypeStruct((1, 1), jnp.int32),
        scratch_shapes=[
            pltpu.SMEM((2,), jnp.int32),
            pltpu.SMEM((_NPROBE,), jnp.int32),
        ],
    )(maxbits, post)

    recon = pl.pallas_call(
        _decode_body,
        grid=(m_blocks,),
        in_specs=[
            pl.BlockSpec(memory_space=pltpu.SMEM),
            pl.BlockSpec((bm, dict_size), lambda m: (m, 0)),
            pl.BlockSpec((act_dim, dict_size), lambda m: (0, 0)),
            pl.BlockSpec((1, act_dim), lambda m: (0, 0)),
        ],
        out_specs=pl.BlockSpec((bm, act_dim), lambda m: (m, 0)),
        out_shape=jax.ShapeDtypeStruct((n_tokens, act_dim), jnp.float32),
    )(theta_bits, post, W_dec, bd2)

    return recon


# speedup vs baseline: 25.5681x; 1.1565x over previous
import functools

import jax
import jax.numpy as jnp
from jax import lax
from jax.experimental import pallas as pl
from jax.experimental.pallas import tpu as pltpu
from jax.experimental.pallas import tpu_sc as plsc

_K_PER_TOKEN = 64

_NC = 2
_NS = 16
_NW = _NC * _NS
_LANES = 16

_CHUNK = 8192
_HI_BINS = 32768
_LO_BINS = 65536


def _encode_body(x_ref, w_ref, be_ref, bd_ref, post_ref):
    xc = x_ref[...] - bd_ref[...]
    acc = lax.dot_general(xc, w_ref[...], (((1,), (1,)), ((), ())),
                          preferred_element_type=jnp.float32)
    post_ref[...] = jnp.maximum(acc + be_ref[...][0], 0.0)


def _sweep_hi_body(total, post_ref, out_ref, buf_ref, hist_ref):
    wid = lax.axis_index("s") * _NC + lax.axis_index("c")
    per_w = total // _NW
    base = wid * per_w

    zeros = jnp.zeros((_LANES,), jnp.int32)

    def zbody(i, _):
        hist_ref[pl.ds(i * _LANES, _LANES)] = zeros
        return 0

    lax.fori_loop(0, _HI_BINS // _LANES, zbody, 0)

    ones = jnp.ones((_LANES,), jnp.int32)

    def chunk_body(g, _):
        pltpu.sync_copy(post_ref.at[pl.ds(base + g * _CHUNK, _CHUNK)],
                        buf_ref)

        def vbody(i, _):
            v = buf_ref[pl.ds(i * _LANES, _LANES)]
            b = lax.bitcast_convert_type(v, jnp.int32)
            idx = lax.shift_right_logical(b, 16)
            plsc.addupdate_scatter(hist_ref, [idx], ones)
            return 0

        lax.fori_loop(0, _CHUNK // _LANES, vbody, 0)
        return 0

    lax.fori_loop(0, per_w // _CHUNK, chunk_body, 0)
    pltpu.sync_copy(hist_ref, out_ref.at[wid])


def _sweep_lo_body(total, post_ref, bvec_ref, out_ref, buf_ref, hist_ref,
                   bv_ref):
    wid = lax.axis_index("s") * _NC + lax.axis_index("c")
    per_w = total // _NW
    base = wid * per_w

    pltpu.sync_copy(bvec_ref.at[pl.ds(0, _LANES)], bv_ref)
    bv = bv_ref[...]

    zeros = jnp.zeros((_LANES,), jnp.int32)

    def zbody(i, _):
        hist_ref[pl.ds(i * _LANES, _LANES)] = zeros
        return 0

    lax.fori_loop(0, _LO_BINS // _LANES, zbody, 0)

    ones = jnp.ones((_LANES,), jnp.int32)

    def chunk_body(g, _):
        pltpu.sync_copy(post_ref.at[pl.ds(base + g * _CHUNK, _CHUNK)],
                        buf_ref)

        def vbody(i, _):
            v = buf_ref[pl.ds(i * _LANES, _LANES)]
            b = lax.bitcast_convert_type(v, jnp.int32)
            hi = lax.shift_right_logical(b, 16)
            lo = jnp.bitwise_and(b, 0xFFFF)
            plsc.addupdate_scatter(hist_ref, [lo], ones, mask=hi == bv)
            return 0

        lax.fori_loop(0, _CHUNK // _LANES, vbody, 0)
        return 0

    lax.fori_loop(0, per_w // _CHUNK, chunk_body, 0)
    pltpu.sync_copy(hist_ref, out_ref.at[wid])


def _suffix_counts(merged, rows, cols):
    m = merged.reshape(rows, cols)
    row_sums = jnp.sum(m, axis=1)
    ii = lax.broadcasted_iota(jnp.int32, (rows, rows), 0)
    jj = lax.broadcasted_iota(jnp.int32, (rows, rows), 1)
    t = jnp.where(jj > ii, row_sums[None, :], 0)
    s_row = jnp.sum(t, axis=1).reshape(rows, 1)
    aa = lax.broadcasted_iota(jnp.int32, (cols, cols), 0)
    bb = lax.broadcasted_iota(jnp.int32, (cols, cols), 1)
    w_incl = (aa >= bb).astype(jnp.float32)
    within = jnp.zeros((rows, cols), jnp.int32)
    for shift in (0, 8, 16):
        plane = lax.shift_right_logical(m, shift)
        if shift < 16:
            plane = jnp.bitwise_and(plane, 255)
        part = lax.dot_general(plane.astype(jnp.float32), w_incl,
                               (((1,), (0,)), ((), ())),
                               preferred_element_type=jnp.float32)
        within = within + part.astype(jnp.int32) * (1 << shift)
    c = within + s_row
    bi = lax.broadcasted_iota(jnp.int32, (rows, cols), 0)
    bl = lax.broadcasted_iota(jnp.int32, (rows, cols), 1)
    return c, bi * cols + bl


def _scan_hi_body(k_total, hist_ref, out_ref):
    merged = jnp.sum(hist_ref[...], axis=0)
    c, jidx = _suffix_counts(merged, _HI_BINS // 128, 128)
    ge = c >= k_total
    bstar = jnp.max(jnp.where(ge, jidx, -1))
    c_next = jnp.max(jnp.where(jidx > bstar, c, 0))
    r = k_total - c_next
    lane = lax.broadcasted_iota(jnp.int32, (1, 128), 1)
    out_ref[...] = jnp.where(lane < _LANES, bstar, r)


def _scan_lo_body(hist_ref, scan_ref, out_ref):
    bstar = scan_ref[0, 0]
    r = scan_ref[0, _LANES]
    merged = jnp.sum(hist_ref[...], axis=0)
    c, jidx = _suffix_counts(merged, _LO_BINS // 128, 128)
    ge = c >= r
    lowstar = jnp.max(jnp.where(ge, jidx, -1))
    theta = jnp.left_shift(bstar, 16) | lowstar
    out_ref[...] = jnp.full((1, 1), theta, dtype=jnp.int32)


def _decode_body(theta_ref, post_ref, w_ref, bd_ref, out_ref):
    th = lax.bitcast_convert_type(theta_ref[0, 0], jnp.float32)
    post = post_ref[...]
    enc = jnp.where(post >= th, post, 0.0)
    acc = lax.dot_general(enc, w_ref[...], (((1,), (1,)), ((), ())),
                          preferred_element_type=jnp.float32)
    out_ref[...] = acc + bd_ref[...]


def kernel(x, W_enc, b_enc, W_dec, b_dec):
    n_tokens, act_dim = x.shape
    dict_size = W_enc.shape[0]
    k_total = _K_PER_TOKEN * n_tokens
    total = n_tokens * dict_size

    bn = min(1024, dict_size)
    n_blocks = dict_size // bn
    bm = min(256, n_tokens)
    m_blocks = n_tokens // bm

    bd2 = b_dec.reshape(1, act_dim)
    be3 = b_enc.reshape(n_blocks, 1, bn)

    post = pl.pallas_call(
        _encode_body,
        grid=(n_blocks,),
        in_specs=[
            pl.BlockSpec((n_tokens, act_dim), lambda n: (0, 0)),
            pl.BlockSpec((bn, act_dim), lambda n: (n, 0)),
            pl.BlockSpec((1, 1, bn), lambda n: (n, 0, 0)),
            pl.BlockSpec((1, act_dim), lambda n: (0, 0)),
        ],
        out_specs=pl.BlockSpec((n_tokens, bn), lambda n: (0, n)),
        out_shape=jax.ShapeDtypeStruct((n_tokens, dict_size), jnp.float32),
    )(x, W_enc, be3, bd2)

    post_flat = post.reshape(total)
    mesh = plsc.VectorSubcoreMesh(core_axis_name="c", subcore_axis_name="s")

    hist_hi = pl.kernel(
        functools.partial(_sweep_hi_body, total),
        mesh=mesh,
        out_type=jax.ShapeDtypeStruct((_NW, _HI_BINS), jnp.int32),
        scratch_types=[
            pltpu.VMEM((_CHUNK,), jnp.float32),
            pltpu.VMEM((_HI_BINS,), jnp.int32),
        ],
        compiler_params=pltpu.CompilerParams(needs_layout_passes=False),
    )(post_flat)

    scan_hi = pl.pallas_call(
        functools.partial(_scan_hi_body, k_total),
        in_specs=[pl.BlockSpec((_NW, _HI_BINS), lambda: (0, 0))],
        out_specs=pl.BlockSpec((1, 128), lambda: (0, 0)),
        out_shape=jax.ShapeDtypeStruct((1, 128), jnp.int32),
    )(hist_hi)

    hist_lo = pl.kernel(
        functools.partial(_sweep_lo_body, total),
        mesh=mesh,
        out_type=jax.ShapeDtypeStruct((_NW, _LO_BINS), jnp.int32),
        scratch_types=[
            pltpu.VMEM((_CHUNK,), jnp.float32),
            pltpu.VMEM((_LO_BINS,), jnp.int32),
            pltpu.VMEM((_LANES,), jnp.int32),
        ],
        compiler_params=pltpu.CompilerParams(needs_layout_passes=False),
    )(post_flat, scan_hi.reshape(128))

    theta_bits = pl.pallas_call(
        _scan_lo_body,
        in_specs=[
            pl.BlockSpec((_NW, _LO_BINS), lambda: (0, 0)),
            pl.BlockSpec(memory_space=pltpu.SMEM),
        ],
        out_specs=pl.BlockSpec((1, 1), lambda: (0, 0)),
        out_shape=jax.ShapeDtypeStruct((1, 1), jnp.int32),
    )(hist_lo, scan_hi)

    recon = pl.pallas_call(
        _decode_body,
        grid=(m_blocks,),
        in_specs=[
            pl.BlockSpec(memory_space=pltpu.SMEM),
            pl.BlockSpec((bm, dict_size), lambda m: (m, 0)),
            pl.BlockSpec((act_dim, dict_size), lambda m: (0, 0)),
            pl.BlockSpec((1, act_dim), lambda m: (0, 0)),
        ],
        out_specs=pl.BlockSpec((bm, act_dim), lambda m: (m, 0)),
        out_shape=jax.ShapeDtypeStruct((n_tokens, act_dim), jnp.float32),
    )(theta_bits, post, W_dec, bd2)

    return recon


# speedup vs baseline: 31.4859x; 1.2315x over previous
import functools

import jax
import jax.numpy as jnp
from jax import lax
from jax.experimental import pallas as pl
from jax.experimental.pallas import tpu as pltpu
from jax.experimental.pallas import tpu_sc as plsc

_K_PER_TOKEN = 64

_NC = 2
_NS = 16
_NW = _NC * _NS
_LANES = 16

_RPC = 2
_UNROLL = 4
_HI_BINS = 32768
_LO_BINS = 65536


def _encode_body(x_ref, w_ref, be_ref, bd_ref, post_ref):
    xc = x_ref[...] - bd_ref[...]
    acc = lax.dot_general(xc, w_ref[...], (((1,), (1,)), ((), ())),
                          preferred_element_type=jnp.float32)
    post_ref[...] = jnp.maximum(acc + be_ref[...][0], 0.0)


def _zero_hist(hist_ref, n_bins):
    zeros = jnp.zeros((_LANES,), jnp.int32)

    def zbody(i, _):
        hist_ref[pl.ds(i * _LANES, _LANES)] = zeros
        return 0

    lax.fori_loop(0, n_bins // _LANES, zbody, 0)


def _sweep_rows(post_ref, base, rpw, buf0, buf1, sem0, sem1, process):
    pltpu.async_copy(post_ref.at[pl.ds(base, _RPC)], buf0, sem0)
    last = base + rpw - _RPC

    def gbody(g, _):
        r0 = base + 2 * _RPC * g
        pltpu.async_copy(post_ref.at[pl.ds(r0 + _RPC, _RPC)], buf1, sem1)
        pltpu.make_async_copy(post_ref.at[pl.ds(0, _RPC)], buf0, sem0).wait()
        process(buf0)
        nxt = jnp.minimum(r0 + 2 * _RPC, last)
        pltpu.async_copy(post_ref.at[pl.ds(nxt, _RPC)], buf0, sem0)
        pltpu.make_async_copy(post_ref.at[pl.ds(0, _RPC)], buf1, sem1).wait()
        process(buf1)
        return 0

    lax.fori_loop(0, rpw // (2 * _RPC), gbody, 0)
    pltpu.make_async_copy(post_ref.at[pl.ds(0, _RPC)], buf0, sem0).wait()


def _sweep_hi_body(n_rows, post_ref, out_ref, buf0, buf1, hist_ref,
                   sem0, sem1):
    wid = lax.axis_index("s") * _NC + lax.axis_index("c")
    rpw = n_rows // _NW
    base = wid * rpw

    _zero_hist(hist_ref, _HI_BINS)
    ones = jnp.ones((_LANES,), jnp.int32)
    n_vec = post_ref.shape[1] // _LANES

    def process(buf):
        for r in range(_RPC):
            def vbody(i, _):
                for u in range(_UNROLL):
                    v = buf[r, pl.ds((i * _UNROLL + u) * _LANES, _LANES)]
                    b = lax.bitcast_convert_type(v, jnp.int32)
                    idx = lax.shift_right_logical(b, 16)
                    plsc.addupdate_scatter(hist_ref, [idx], ones)
                return 0

            lax.fori_loop(0, n_vec // _UNROLL, vbody, 0)

    _sweep_rows(post_ref, base, rpw, buf0, buf1, sem0, sem1, process)
    pltpu.sync_copy(hist_ref, out_ref.at[wid])


def _sweep_lo_body(n_rows, post_ref, bvec_ref, out_ref, buf0, buf1,
                   hist_ref, bv_ref, sem0, sem1):
    wid = lax.axis_index("s") * _NC + lax.axis_index("c")
    rpw = n_rows // _NW
    base = wid * rpw

    pltpu.sync_copy(bvec_ref.at[pl.ds(0, _LANES)], bv_ref)
    bv = bv_ref[...]

    _zero_hist(hist_ref, _LO_BINS)
    ones = jnp.ones((_LANES,), jnp.int32)
    n_vec = post_ref.shape[1] // _LANES

    def process(buf):
        for r in range(_RPC):
            def vbody(i, _):
                for u in range(_UNROLL):
                    v = buf[r, pl.ds((i * _UNROLL + u) * _LANES, _LANES)]
                    b = lax.bitcast_convert_type(v, jnp.int32)
                    hi = lax.shift_right_logical(b, 16)
                    lo = jnp.bitwise_and(b, 0xFFFF)
                    plsc.addupdate_scatter(hist_ref, [lo], ones,
                                           mask=hi == bv)
                return 0

            lax.fori_loop(0, n_vec // _UNROLL, vbody, 0)

    _sweep_rows(post_ref, base, rpw, buf0, buf1, sem0, sem1, process)
    pltpu.sync_copy(hist_ref, out_ref.at[wid])


def _suffix_counts(merged, rows, cols):
    m = merged.reshape(rows, cols)
    row_sums = jnp.sum(m, axis=1)
    ii = lax.broadcasted_iota(jnp.int32, (rows, rows), 0)
    jj = lax.broadcasted_iota(jnp.int32, (rows, rows), 1)
    t = jnp.where(jj > ii, row_sums[None, :], 0)
    s_row = jnp.sum(t, axis=1).reshape(rows, 1)
    aa = lax.broadcasted_iota(jnp.int32, (cols, cols), 0)
    bb = lax.broadcasted_iota(jnp.int32, (cols, cols), 1)
    w_incl = (aa >= bb).astype(jnp.float32)
    within = jnp.zeros((rows, cols), jnp.int32)
    for shift in (0, 8, 16):
        plane = lax.shift_right_logical(m, shift)
        if shift < 16:
            plane = jnp.bitwise_and(plane, 255)
        part = lax.dot_general(plane.astype(jnp.float32), w_incl,
                               (((1,), (0,)), ((), ())),
                               preferred_element_type=jnp.float32)
        within = within + part.astype(jnp.int32) * (1 << shift)
    c = within + s_row
    bi = lax.broadcasted_iota(jnp.int32, (rows, cols), 0)
    bl = lax.broadcasted_iota(jnp.int32, (rows, cols), 1)
    return c, bi * cols + bl


def _scan_hi_body(k_total, hist_ref, out_ref):
    merged = jnp.sum(hist_ref[...], axis=0)
    c, jidx = _suffix_counts(merged, _HI_BINS // 128, 128)
    ge = c >= k_total
    bstar = jnp.max(jnp.where(ge, jidx, -1))
    c_next = jnp.max(jnp.where(jidx > bstar, c, 0))
    r = k_total - c_next
    lane = lax.broadcasted_iota(jnp.int32, (1, 128), 1)
    out_ref[...] = jnp.where(lane < _LANES, bstar, r)


def _scan_lo_body(hist_ref, scan_ref, out_ref):
    bstar = scan_ref[0, 0]
    r = scan_ref[0, _LANES]
    merged = jnp.sum(hist_ref[...], axis=0)
    c, jidx = _suffix_counts(merged, _LO_BINS // 128, 128)
    ge = c >= r
    lowstar = jnp.max(jnp.where(ge, jidx, -1))
    theta = jnp.left_shift(bstar, 16) | lowstar
    out_ref[...] = jnp.full((1, 1), theta, dtype=jnp.int32)


def _decode_body(theta_ref, post_ref, w_ref, bd_ref, out_ref):
    th = lax.bitcast_convert_type(theta_ref[0, 0], jnp.float32)
    post = post_ref[...]
    enc = jnp.where(post >= th, post, 0.0)
    acc = lax.dot_general(enc, w_ref[...], (((1,), (1,)), ((), ())),
                          preferred_element_type=jnp.float32)
    out_ref[...] = acc + bd_ref[...]


def kernel(x, W_enc, b_enc, W_dec, b_dec):
    n_tokens, act_dim = x.shape
    dict_size = W_enc.shape[0]
    k_total = _K_PER_TOKEN * n_tokens
    total = n_tokens * dict_size

    bn = min(1024, dict_size)
    n_blocks = dict_size // bn
    bm = min(256, n_tokens)
    m_blocks = n_tokens // bm

    bd2 = b_dec.reshape(1, act_dim)
    be3 = b_enc.reshape(n_blocks, 1, bn)

    post = pl.pallas_call(
        _encode_body,
        grid=(n_blocks,),
        in_specs=[
            pl.BlockSpec((n_tokens, act_dim), lambda n: (0, 0)),
            pl.BlockSpec((bn, act_dim), lambda n: (n, 0)),
            pl.BlockSpec((1, 1, bn), lambda n: (n, 0, 0)),
            pl.BlockSpec((1, act_dim), lambda n: (0, 0)),
        ],
        out_specs=pl.BlockSpec((n_tokens, bn), lambda n: (0, n)),
        out_shape=jax.ShapeDtypeStruct((n_tokens, dict_size), jnp.float32),
    )(x, W_enc, be3, bd2)

    mesh = plsc.VectorSubcoreMesh(core_axis_name="c", subcore_axis_name="s")

    hist_hi = pl.kernel(
        functools.partial(_sweep_hi_body, n_tokens),
        mesh=mesh,
        out_type=jax.ShapeDtypeStruct((_NW, _HI_BINS), jnp.int32),
        scratch_types=[
            pltpu.VMEM((_RPC, dict_size), jnp.float32),
            pltpu.VMEM((_RPC, dict_size), jnp.float32),
            pltpu.VMEM((_HI_BINS,), jnp.int32),
            pltpu.SemaphoreType.DMA,
            pltpu.SemaphoreType.DMA,
        ],
        compiler_params=pltpu.CompilerParams(needs_layout_passes=False),
    )(post)

    scan_hi = pl.pallas_call(
        functools.partial(_scan_hi_body, k_total),
        in_specs=[pl.BlockSpec((_NW, _HI_BINS), lambda: (0, 0))],
        out_specs=pl.BlockSpec((1, 128), lambda: (0, 0)),
        out_shape=jax.ShapeDtypeStruct((1, 128), jnp.int32),
    )(hist_hi)

    hist_lo = pl.kernel(
        functools.partial(_sweep_lo_body, n_tokens),
        mesh=mesh,
        out_type=jax.ShapeDtypeStruct((_NW, _LO_BINS), jnp.int32),
        scratch_types=[
            pltpu.VMEM((_RPC, dict_size), jnp.float32),
            pltpu.VMEM((_RPC, dict_size), jnp.float32),
            pltpu.VMEM((_LO_BINS,), jnp.int32),
            pltpu.VMEM((_LANES,), jnp.int32),
            pltpu.SemaphoreType.DMA,
            pltpu.SemaphoreType.DMA,
        ],
        compiler_params=pltpu.CompilerParams(needs_layout_passes=False),
    )(post, scan_hi.reshape(128))

    theta_bits = pl.pallas_call(
        _scan_lo_body,
        in_specs=[
            pl.BlockSpec((_NW, _LO_BINS), lambda: (0, 0)),
            pl.BlockSpec(memory_space=pltpu.SMEM),
        ],
        out_specs=pl.BlockSpec((1, 1), lambda: (0, 0)),
        out_shape=jax.ShapeDtypeStruct((1, 1), jnp.int32),
    )(hist_lo, scan_hi)

    recon = pl.pallas_call(
        _decode_body,
        grid=(m_blocks,),
        in_specs=[
            pl.BlockSpec(memory_space=pltpu.SMEM),
            pl.BlockSpec((bm, dict_size), lambda m: (m, 0)),
            pl.BlockSpec((act_dim, dict_size), lambda m: (0, 0)),
            pl.BlockSpec((1, act_dim), lambda m: (0, 0)),
        ],
        out_specs=pl.BlockSpec((bm, act_dim), lambda m: (m, 0)),
        out_shape=jax.ShapeDtypeStruct((n_tokens, act_dim), jnp.float32),
    )(theta_bits, post, W_dec, bd2)

    return recon


# speedup vs baseline: 62.8486x; 1.9961x over previous
import functools

import jax
import jax.numpy as jnp
from jax import lax
from jax.experimental import pallas as pl
from jax.experimental.pallas import tpu as pltpu
from jax.experimental.pallas import tpu_sc as plsc

_K_PER_TOKEN = 64

_NC = 2
_NS = 16
_NW = _NC * _NS
_LANES = 16

_RPC = 2
_UNROLL = 4
_HI_BINS = 32768
_LO_BINS = 65536


def _encode_body(x_ref, w_ref, be_ref, bd_ref, post_ref):
    xc = x_ref[...] - bd_ref[...]
    acc = lax.dot_general(xc, w_ref[...], (((1,), (1,)), ((), ())),
                          preferred_element_type=jnp.float32)
    post_ref[...] = jnp.maximum(acc + be_ref[...][0], 0.0)


def _zero_hist(hist_ref, n_bins):
    zeros = jnp.zeros((_LANES,), jnp.int32)

    def zbody(i, _):
        hist_ref[pl.ds(i * _LANES, _LANES)] = zeros
        return 0

    lax.fori_loop(0, n_bins // _LANES, zbody, 0)


def _sweep_rows(post_ref, base, rpw, buf0, buf1, sem0, sem1, process):
    pltpu.async_copy(post_ref.at[pl.ds(base, _RPC)], buf0, sem0)
    last = base + rpw - _RPC

    def gbody(g, _):
        r0 = base + 2 * _RPC * g
        pltpu.async_copy(post_ref.at[pl.ds(r0 + _RPC, _RPC)], buf1, sem1)
        pltpu.make_async_copy(post_ref.at[pl.ds(0, _RPC)], buf0, sem0).wait()
        process(buf0)
        nxt = jnp.minimum(r0 + 2 * _RPC, last)
        pltpu.async_copy(post_ref.at[pl.ds(nxt, _RPC)], buf0, sem0)
        pltpu.make_async_copy(post_ref.at[pl.ds(0, _RPC)], buf1, sem1).wait()
        process(buf1)
        return 0

    lax.fori_loop(0, rpw // (2 * _RPC), gbody, 0)
    pltpu.make_async_copy(post_ref.at[pl.ds(0, _RPC)], buf0, sem0).wait()


def _sweep_hi_body(n_rows, post_ref, out_ref, buf0, buf1, hist_ref,
                   sem0, sem1):
    wid = lax.axis_index("s") * _NC + lax.axis_index("c")
    rpw = n_rows // _NW
    base = wid * rpw

    _zero_hist(hist_ref, _HI_BINS)
    ones = jnp.ones((_LANES,), jnp.int32)
    n_vec = post_ref.shape[1] // _LANES

    def process(buf):
        for r in range(_RPC):
            @plsc.parallel_loop(0, n_vec, step=1, unroll=_UNROLL)
            def vbody(i):
                v = buf[r, pl.ds(i * _LANES, _LANES)]
                b = lax.bitcast_convert_type(v, jnp.int32)
                idx = lax.shift_right_logical(b, 16)
                plsc.addupdate_scatter(hist_ref, [idx], ones)

    _sweep_rows(post_ref, base, rpw, buf0, buf1, sem0, sem1, process)
    pltpu.sync_copy(hist_ref, out_ref.at[wid])


def _sweep_lo_body(n_rows, post_ref, bvec_ref, out_ref, buf0, buf1,
                   hist_ref, bv_ref, sem0, sem1):
    wid = lax.axis_index("s") * _NC + lax.axis_index("c")
    rpw = n_rows // _NW
    base = wid * rpw

    pltpu.sync_copy(bvec_ref.at[pl.ds(0, _LANES)], bv_ref)
    bv = bv_ref[...]

    _zero_hist(hist_ref, _LO_BINS)
    ones = jnp.ones((_LANES,), jnp.int32)
    n_vec = post_ref.shape[1] // _LANES

    def process(buf):
        for r in range(_RPC):
            @plsc.parallel_loop(0, n_vec, step=1, unroll=_UNROLL)
            def vbody(i):
                v = buf[r, pl.ds(i * _LANES, _LANES)]
                b = lax.bitcast_convert_type(v, jnp.int32)
                hi = lax.shift_right_logical(b, 16)
                lo = jnp.bitwise_and(b, 0xFFFF)
                plsc.addupdate_scatter(hist_ref, [lo], ones, mask=hi == bv)

    _sweep_rows(post_ref, base, rpw, buf0, buf1, sem0, sem1, process)
    pltpu.sync_copy(hist_ref, out_ref.at[wid])


def _suffix_counts(merged, rows, cols):
    m = merged.reshape(rows, cols)
    row_sums = jnp.sum(m, axis=1)
    ii = lax.broadcasted_iota(jnp.int32, (rows, rows), 0)
    jj = lax.broadcasted_iota(jnp.int32, (rows, rows), 1)
    t = jnp.where(jj > ii, row_sums[None, :], 0)
    s_row = jnp.sum(t, axis=1).reshape(rows, 1)
    aa = lax.broadcasted_iota(jnp.int32, (cols, cols), 0)
    bb = lax.broadcasted_iota(jnp.int32, (cols, cols), 1)
    w_incl = (aa >= bb).astype(jnp.float32)
    within = jnp.zeros((rows, cols), jnp.int32)
    for shift in (0, 8, 16):
        plane = lax.shift_right_logical(m, shift)
        if shift < 16:
            plane = jnp.bitwise_and(plane, 255)
        part = lax.dot_general(plane.astype(jnp.float32), w_incl,
                               (((1,), (0,)), ((), ())),
                               preferred_element_type=jnp.float32)
        within = within + part.astype(jnp.int32) * (1 << shift)
    c = within + s_row
    bi = lax.broadcasted_iota(jnp.int32, (rows, cols), 0)
    bl = lax.broadcasted_iota(jnp.int32, (rows, cols), 1)
    return c, bi * cols + bl


def _scan_hi_body(k_total, hist_ref, out_ref):
    merged = jnp.sum(hist_ref[...], axis=0)
    c, jidx = _suffix_counts(merged, _HI_BINS // 128, 128)
    ge = c >= k_total
    bstar = jnp.max(jnp.where(ge, jidx, -1))
    c_next = jnp.max(jnp.where(jidx > bstar, c, 0))
    r = k_total - c_next
    lane = lax.broadcasted_iota(jnp.int32, (1, 128), 1)
    out_ref[...] = jnp.where(lane < _LANES, bstar, r)


def _scan_lo_body(hist_ref, scan_ref, out_ref):
    bstar = scan_ref[0, 0]
    r = scan_ref[0, _LANES]
    merged = jnp.sum(hist_ref[...], axis=0)
    c, jidx = _suffix_counts(merged, _LO_BINS // 128, 128)
    ge = c >= r
    lowstar = jnp.max(jnp.where(ge, jidx, -1))
    theta = jnp.left_shift(bstar, 16) | lowstar
    out_ref[...] = jnp.full((1, 1), theta, dtype=jnp.int32)


def _decode_body(theta_ref, post_ref, w_ref, bd_ref, out_ref):
    th = lax.bitcast_convert_type(theta_ref[0, 0], jnp.float32)
    post = post_ref[...]
    enc = jnp.where(post >= th, post, 0.0)
    acc = lax.dot_general(enc, w_ref[...], (((1,), (1,)), ((), ())),
                          preferred_element_type=jnp.float32)
    out_ref[...] = acc + bd_ref[...]


def kernel(x, W_enc, b_enc, W_dec, b_dec):
    n_tokens, act_dim = x.shape
    dict_size = W_enc.shape[0]
    k_total = _K_PER_TOKEN * n_tokens
    total = n_tokens * dict_size

    bn = min(1024, dict_size)
    n_blocks = dict_size // bn
    bm = min(256, n_tokens)
    m_blocks = n_tokens // bm

    bd2 = b_dec.reshape(1, act_dim)
    be3 = b_enc.reshape(n_blocks, 1, bn)

    post = pl.pallas_call(
        _encode_body,
        grid=(n_blocks,),
        in_specs=[
            pl.BlockSpec((n_tokens, act_dim), lambda n: (0, 0)),
            pl.BlockSpec((bn, act_dim), lambda n: (n, 0)),
            pl.BlockSpec((1, 1, bn), lambda n: (n, 0, 0)),
            pl.BlockSpec((1, act_dim), lambda n: (0, 0)),
        ],
        out_specs=pl.BlockSpec((n_tokens, bn), lambda n: (0, n)),
        out_shape=jax.ShapeDtypeStruct((n_tokens, dict_size), jnp.float32),
    )(x, W_enc, be3, bd2)

    mesh = plsc.VectorSubcoreMesh(core_axis_name="c", subcore_axis_name="s")

    hist_hi = pl.kernel(
        functools.partial(_sweep_hi_body, n_tokens),
        mesh=mesh,
        out_type=jax.ShapeDtypeStruct((_NW, _HI_BINS), jnp.int32),
        scratch_types=[
            pltpu.VMEM((_RPC, dict_size), jnp.float32),
            pltpu.VMEM((_RPC, dict_size), jnp.float32),
            pltpu.VMEM((_HI_BINS,), jnp.int32),
            pltpu.SemaphoreType.DMA,
            pltpu.SemaphoreType.DMA,
        ],
        compiler_params=pltpu.CompilerParams(needs_layout_passes=False),
    )(post)

    scan_hi = pl.pallas_call(
        functools.partial(_scan_hi_body, k_total),
        in_specs=[pl.BlockSpec((_NW, _HI_BINS), lambda: (0, 0))],
        out_specs=pl.BlockSpec((1, 128), lambda: (0, 0)),
        out_shape=jax.ShapeDtypeStruct((1, 128), jnp.int32),
    )(hist_hi)

    hist_lo = pl.kernel(
        functools.partial(_sweep_lo_body, n_tokens),
        mesh=mesh,
        out_type=jax.ShapeDtypeStruct((_NW, _LO_BINS), jnp.int32),
        scratch_types=[
            pltpu.VMEM((_RPC, dict_size), jnp.float32),
            pltpu.VMEM((_RPC, dict_size), jnp.float32),
            pltpu.VMEM((_LO_BINS,), jnp.int32),
            pltpu.VMEM((_LANES,), jnp.int32),
            pltpu.SemaphoreType.DMA,
            pltpu.SemaphoreType.DMA,
        ],
        compiler_params=pltpu.CompilerParams(needs_layout_passes=False),
    )(post, scan_hi.reshape(128))

    theta_bits = pl.pallas_call(
        _scan_lo_body,
        in_specs=[
            pl.BlockSpec((_NW, _LO_BINS), lambda: (0, 0)),
            pl.BlockSpec(memory_space=pltpu.SMEM),
        ],
        out_specs=pl.BlockSpec((1, 1), lambda: (0, 0)),
        out_shape=jax.ShapeDtypeStruct((1, 1), jnp.int32),
    )(hist_lo, scan_hi)

    recon = pl.pallas_call(
        _decode_body,
        grid=(m_blocks,),
        in_specs=[
            pl.BlockSpec(memory_space=pltpu.SMEM),
            pl.BlockSpec((bm, dict_size), lambda m: (m, 0)),
            pl.BlockSpec((act_dim, dict_size), lambda m: (0, 0)),
            pl.BlockSpec((1, act_dim), lambda m: (0, 0)),
        ],
        out_specs=pl.BlockSpec((bm, act_dim), lambda m: (m, 0)),
        out_shape=jax.ShapeDtypeStruct((n_tokens, act_dim), jnp.float32),
    )(theta_bits, post, W_dec, bd2)

    return recon
